# Initial kernel scaffold; baseline (speedup 1.0000x reference)
import jax
import jax.numpy as jnp
from jax.experimental import pallas as pl


def kernel(x, edge, WQ, WV, WK, a):
    raise NotImplementedError("write your pallas kernel here")


# speedup vs baseline: 71.0708x; 71.0708x over previous
import functools
import math

import jax
import jax.numpy as jnp
from jax import lax
from jax.experimental import pallas as pl
from jax.experimental.pallas import tpu as pltpu
from jax.experimental.pallas import tpu_sc as plsc

N_NODES = 10000
N_EDGES = 320000
F = 128
K = 10
NITER = 10
KPAD = 128

NTILES = 16
ROWS_PER_TILE = 160
NROWS = NTILES * ROWS_PER_TILE
E_PAD = NROWS * 128
NPAD = 10240
ZCHUNK = NPAD // NTILES


def _tc_body(x_ref, c0_ref, wq_ref, wv_ref, alloc_ref, et_ref):
    x = x_ref[...]
    col = lax.broadcasted_iota(jnp.int32, (1, KPAD), 1)
    row = lax.broadcasted_iota(jnp.int32, (KPAD, 1), 0)
    valid = col < K
    ones_col = jnp.ones((N_NODES, 1), jnp.float32)
    xa = jnp.concatenate([x, ones_col], axis=1)

    def assign(c):
        cn = jnp.sum(c * c, axis=1, keepdims=True)
        b = jnp.concatenate([c, -0.5 * cn], axis=1)
        dots = lax.dot_general(xa, b, (((1,), (1,)), ((), ())),
                               preferred_element_type=jnp.float32)
        score = jnp.where(valid, -2.0 * dots, 1e30)
        m = jnp.min(score, axis=1, keepdims=True)
        return jnp.min(jnp.where(score == m, col, KPAD), axis=1,
                       keepdims=True)

    def update(cl):
        onehot = (cl == col).astype(jnp.float32)
        sa = lax.dot_general(onehot, xa, (((0,), (0,)), ((), ())),
                             preferred_element_type=jnp.float32)
        return jnp.where(row < K, sa[:, :F] / sa[:, F:F + 1], 0.0)

    c9 = lax.fori_loop(0, NITER - 1, lambda i, c: update(assign(c)),
                       c0_ref[...])
    cl = assign(c9)
    c10 = update(cl)

    q = jnp.dot(c10, wq_ref[...], preferred_element_type=jnp.float32)
    v = jnp.dot(c10, wv_ref[...], preferred_element_type=jnp.float32)
    prods = lax.dot_general(q, v, (((1,), (1,)), ((), ())),
                            preferred_element_type=jnp.float32)
    prods = prods * (1.0 / math.sqrt(F))
    p = jnp.where(valid, prods, -1e30)
    m = jnp.max(p, axis=1, keepdims=True)
    e = jnp.exp(p - m)
    tab = e / jnp.sum(e, axis=1, keepdims=True)

    alloc_ref[...] = cl
    et_ref[...] = jnp.exp(tab)


_tc_call = pl.pallas_call(
    _tc_body,
    out_shape=[
        jax.ShapeDtypeStruct((N_NODES, 1), jnp.int32),
        jax.ShapeDtypeStruct((KPAD, KPAD), jnp.float32),
    ],
)


def _sc_body(src_hbm, dst_hbm, alloc_hbm, et_hbm, zeros_hbm, out_hbm,
             src_v, dst_v, w_v, alloc_v, et_v, s_v, z_v, s_sh, dsem):
    sid = lax.axis_index("s")
    row0 = sid * ROWS_PER_TILE

    pltpu.sync_copy(src_hbm.at[pl.ds(row0, ROWS_PER_TILE)], src_v)
    pltpu.sync_copy(dst_hbm.at[pl.ds(row0, ROWS_PER_TILE)], dst_v)
    pltpu.sync_copy(alloc_hbm, alloc_v)
    pltpu.sync_copy(et_hbm, et_v)
    pltpu.sync_copy(zeros_hbm.at[pl.ds(sid * ZCHUNK, ZCHUNK)], z_v)
    pltpu.sync_copy(z_v, s_sh.at[pl.ds(sid * ZCHUNK, ZCHUNK)])
    plsc.subcore_barrier()

    ebase = row0 * 128
    lane = lax.iota(jnp.int32, 16)

    def compute_row(j, carry):
        for k in range(8):
            off = k * 16
            s16 = src_v[j, pl.ds(off, 16)]
            d16 = dst_v[j, pl.ds(off, 16)]
            cs = plsc.load_gather(alloc_v, [s16])
            cd = plsc.load_gather(alloc_v, [d16])
            w16 = plsc.load_gather(et_v, [cs * K + cd])
            gidx = ebase + j * 128 + off + lane
            w_v[j, pl.ds(off, 16)] = jnp.where(gidx < N_EDGES, w16, 0.0)
        return carry

    lax.fori_loop(0, ROWS_PER_TILE, compute_row, 0)

    def scat_chunk(g, carry):
        r = g * 8
        for b in range(8):
            pltpu.async_copy(w_v.at[r + b], s_sh.at[src_v.at[r + b]], dsem,
                             add=True)
        for b in range(8):
            pltpu.make_async_copy(w_v.at[r + b], s_sh.at[src_v.at[r + b]],
                                  dsem).wait()
        return carry

    lax.fori_loop(0, ROWS_PER_TILE // 8, scat_chunk, 0)
    plsc.subcore_barrier()
    pltpu.sync_copy(s_sh, s_v)

    def div_row(j, carry):
        for k in range(8):
            off = k * 16
            s16 = src_v[j, pl.ds(off, 16)]
            seg = plsc.load_gather(s_v, [s16])
            w_v[j, pl.ds(off, 16)] = w_v[j, pl.ds(off, 16)] / seg
        return carry

    lax.fori_loop(0, ROWS_PER_TILE, div_row, 0)
    pltpu.sync_copy(w_v, out_hbm.at[pl.ds(row0, ROWS_PER_TILE)])


_sc_call = functools.partial(
    pl.kernel,
    mesh=plsc.VectorSubcoreMesh(core_axis_name="c", subcore_axis_name="s",
                                num_cores=1),
    compiler_params=pltpu.CompilerParams(needs_layout_passes=False),
    out_type=jax.ShapeDtypeStruct((NROWS, 128), jnp.float32),
    scratch_types=[
        pltpu.VMEM((ROWS_PER_TILE, 128), jnp.int32),
        pltpu.VMEM((ROWS_PER_TILE, 128), jnp.int32),
        pltpu.VMEM((ROWS_PER_TILE, 128), jnp.float32),
        pltpu.VMEM((N_NODES,), jnp.int32),
        pltpu.VMEM((128,), jnp.float32),
        pltpu.VMEM((NPAD,), jnp.float32),
        pltpu.VMEM((ZCHUNK,), jnp.float32),
        pltpu.VMEM_SHARED((NPAD,), jnp.float32),
        pltpu.SemaphoreType.DMA,
    ],
)(_sc_body)


def kernel(x, edge, WQ, WV, WK, a):
    c0 = jnp.concatenate(
        [x[:K], jnp.zeros((KPAD - K, F), jnp.float32)], axis=0)
    alloc2, et_full = _tc_call(x, c0, WQ, WV)
    alloc = alloc2.reshape(N_NODES)
    et_flat = jnp.pad(et_full[:K, :K].reshape(K * K), (0, 128 - K * K))
    srcp = jnp.pad(edge[0], (0, E_PAD - N_EDGES)).reshape(NROWS, 128)
    dstp = jnp.pad(edge[1], (0, E_PAD - N_EDGES)).reshape(NROWS, 128)
    zeros = jnp.zeros((NPAD,), jnp.float32)
    outp = _sc_call(srcp, dstp, alloc, et_flat, zeros)
    return outp.reshape(E_PAD)[:N_EDGES]


# speedup vs baseline: 78.0037x; 1.0975x over previous
import functools
import math

import jax
import jax.numpy as jnp
from jax import lax
from jax.experimental import pallas as pl
from jax.experimental.pallas import tpu as pltpu
from jax.experimental.pallas import tpu_sc as plsc

N_NODES = 10000
N_EDGES = 320000
F = 128
K = 10
NITER = 10
KPAD = 128

NTILES = 16
NROWS = N_EDGES // 128
R_BIG = 160
R_LAST = NROWS - 15 * R_BIG
NPAD = 10240
ZCHUNK = NPAD // NTILES
NBUF = 16


def _tc_body(x_ref, c0_ref, wq_ref, wv_ref, alloc_ref, alloc16_ref, et_ref):
    x = x_ref[...]
    col = lax.broadcasted_iota(jnp.int32, (1, KPAD), 1)
    row = lax.broadcasted_iota(jnp.int32, (KPAD, 1), 0)
    valid = col < K
    ones_col = jnp.ones((N_NODES, 1), jnp.float32)
    xa = jnp.concatenate([x, ones_col], axis=1)

    def assign(c):
        cn = jnp.sum(c * c, axis=1, keepdims=True)
        b = jnp.concatenate([c, -0.5 * cn], axis=1)
        dots = lax.dot_general(xa, b, (((1,), (1,)), ((), ())),
                               preferred_element_type=jnp.float32)
        score = jnp.where(valid, -2.0 * dots, 1e30)
        m = jnp.min(score, axis=1, keepdims=True)
        return jnp.min(jnp.where(score == m, col, KPAD), axis=1,
                       keepdims=True)

    def update(cl):
        onehot = (cl == col).astype(jnp.float32)
        sa = lax.dot_general(onehot, xa, (((0,), (0,)), ((), ())),
                             preferred_element_type=jnp.float32)
        return jnp.where(row < K, sa[:, :F] / sa[:, F:F + 1], 0.0)

    c9 = lax.fori_loop(0, NITER - 1, lambda i, c: update(assign(c)),
                       c0_ref[...])
    cl = assign(c9)
    c10 = update(cl)

    q = jnp.dot(c10, wq_ref[...], preferred_element_type=jnp.float32)
    v = jnp.dot(c10, wv_ref[...], preferred_element_type=jnp.float32)
    prods = lax.dot_general(q, v, (((1,), (1,)), ((), ())),
                            preferred_element_type=jnp.float32)
    prods = prods * (1.0 / math.sqrt(F))
    p = jnp.where(valid, prods, -1e30)
    m = jnp.max(p, axis=1, keepdims=True)
    e = jnp.exp(p - m)
    tab = e / jnp.sum(e, axis=1, keepdims=True)

    alloc_ref[...] = cl
    alloc16_ref[...] = cl * 16
    et_ref[...] = jnp.exp(tab[:K, :16])


_tc_call = pl.pallas_call(
    _tc_body,
    out_shape=[
        jax.ShapeDtypeStruct((N_NODES, 1), jnp.int32),
        jax.ShapeDtypeStruct((N_NODES, 1), jnp.int32),
        jax.ShapeDtypeStruct((K, 16), jnp.float32),
    ],
)


def _sc_body(src_hbm, dst_hbm, alloc_hbm, alloc16_hbm, et_hbm, out_hbm,
             src_v, dst_v, w_v, alloc_v, alloc16_v, et_v, s_v, z_v, s_sh,
             dsem, lsem):
    sid = lax.axis_index("s")
    row0 = pl.multiple_of(sid * R_BIG, 8)
    is_last = sid == NTILES - 1
    nrows = jnp.where(is_last, R_LAST, R_BIG)

    cps = [
        pltpu.async_copy(alloc_hbm, alloc_v, lsem),
        pltpu.async_copy(alloc16_hbm, alloc16_v, lsem),
        pltpu.async_copy(et_hbm, et_v, lsem),
    ]

    @pl.when(jnp.logical_not(is_last))
    def _():
        c1 = pltpu.async_copy(src_hbm.at[pl.ds(row0, R_BIG)],
                              src_v.at[pl.ds(0, R_BIG)], lsem)
        c2 = pltpu.async_copy(dst_hbm.at[pl.ds(row0, R_BIG)],
                              dst_v.at[pl.ds(0, R_BIG)], lsem)
        c1.wait()
        c2.wait()

    @pl.when(is_last)
    def _():
        c1 = pltpu.async_copy(src_hbm.at[pl.ds(15 * R_BIG, R_LAST)],
                              src_v.at[pl.ds(0, R_LAST)], lsem)
        c2 = pltpu.async_copy(dst_hbm.at[pl.ds(15 * R_BIG, R_LAST)],
                              dst_v.at[pl.ds(0, R_LAST)], lsem)
        c1.wait()
        c2.wait()

    for t in range(ZCHUNK // 16):
        z_v[pl.ds(t * 16, 16)] = jnp.zeros((16,), jnp.float32)
    for c in cps:
        c.wait()
    pltpu.sync_copy(z_v, s_sh.at[pl.ds(sid * ZCHUNK, ZCHUNK)])
    plsc.subcore_barrier()

    def compute_row(j, carry):
        @pl.when(j >= NBUF)
        def _():
            jj = j - NBUF
            pltpu.make_async_copy(w_v.at[jj], s_sh.at[src_v.at[jj]],
                                  dsem).wait()
        for k in range(8):
            off = k * 16
            s16 = src_v[j, pl.ds(off, 16)]
            d16 = dst_v[j, pl.ds(off, 16)]
            cs16 = plsc.load_gather(alloc16_v, [s16])
            cd = plsc.load_gather(alloc_v, [d16])
            w_v[j, pl.ds(off, 16)] = plsc.load_gather(et_v, [cs16 + cd])
        pltpu.async_copy(w_v.at[j], s_sh.at[src_v.at[j]], dsem, add=True)
        return carry

    lax.fori_loop(0, nrows, compute_row, 0)

    def drain_row(j, carry):
        pltpu.make_async_copy(w_v.at[j], s_sh.at[src_v.at[j]], dsem).wait()
        return carry

    lax.fori_loop(nrows - NBUF, nrows, drain_row, 0)
    plsc.subcore_barrier()

    pltpu.sync_copy(s_sh.at[pl.ds(sid * ZCHUNK, ZCHUNK)], z_v)
    for t in range(ZCHUNK // 16):
        z_v[pl.ds(t * 16, 16)] = 1.0 / z_v[pl.ds(t * 16, 16)]
    pltpu.sync_copy(z_v, s_sh.at[pl.ds(sid * ZCHUNK, ZCHUNK)])
    plsc.subcore_barrier()
    pltpu.sync_copy(s_sh, s_v)

    def div_row(j, carry):
        for k in range(8):
            off = k * 16
            s16 = src_v[j, pl.ds(off, 16)]
            inv = plsc.load_gather(s_v, [s16])
            w_v[j, pl.ds(off, 16)] = w_v[j, pl.ds(off, 16)] * inv
        return carry

    lax.fori_loop(0, nrows, div_row, 0)

    @pl.when(jnp.logical_not(is_last))
    def _():
        pltpu.sync_copy(w_v.at[pl.ds(0, R_BIG)],
                        out_hbm.at[pl.ds(row0, R_BIG)])

    @pl.when(is_last)
    def _():
        pltpu.sync_copy(w_v.at[pl.ds(0, R_LAST)],
                        out_hbm.at[pl.ds(15 * R_BIG, R_LAST)])


_sc_call = pl.kernel(
    _sc_body,
    mesh=plsc.VectorSubcoreMesh(core_axis_name="c", subcore_axis_name="s",
                                num_cores=1),
    compiler_params=pltpu.CompilerParams(needs_layout_passes=False),
    out_type=jax.ShapeDtypeStruct((NROWS, 128), jnp.float32),
    scratch_types=[
        pltpu.VMEM((R_BIG, 128), jnp.int32),
        pltpu.VMEM((R_BIG, 128), jnp.int32),
        pltpu.VMEM((R_BIG, 128), jnp.float32),
        pltpu.VMEM((N_NODES,), jnp.int32),
        pltpu.VMEM((N_NODES,), jnp.int32),
        pltpu.VMEM((K * 16,), jnp.float32),
        pltpu.VMEM((NPAD,), jnp.float32),
        pltpu.VMEM((ZCHUNK,), jnp.float32),
        pltpu.VMEM_SHARED((NPAD,), jnp.float32),
        pltpu.SemaphoreType.DMA,
        pltpu.SemaphoreType.DMA,
    ],
)


def kernel(x, edge, WQ, WV, WK, a):
    c0 = jnp.concatenate(
        [x[:K], jnp.zeros((KPAD - K, F), jnp.float32)], axis=0)
    alloc2, alloc16_2, et10 = _tc_call(x, c0, WQ, WV)
    outp = _sc_call(edge[0].reshape(NROWS, 128), edge[1].reshape(NROWS, 128),
                    alloc2.reshape(N_NODES), alloc16_2.reshape(N_NODES),
                    et10.reshape(K * 16))
    return outp.reshape(N_EDGES)


# speedup vs baseline: 112.3153x; 1.4399x over previous
import functools
import math

import jax
import jax.numpy as jnp
from jax import lax
from jax.experimental import pallas as pl
from jax.experimental.pallas import tpu as pltpu
from jax.experimental.pallas import tpu_sc as plsc

N_NODES = 10000
N_EDGES = 320000
F = 128
K = 10
NITER = 10
KPAD = 128

NTILES = 16
NROWS = N_EDGES // 128
R_BIG = 160
R_LAST = NROWS - 15 * R_BIG
NPAD = 10240
ZCHUNK = NPAD // NTILES
NBUF = 16


def _tc_body(x_ref, c0_ref, wq_ref, wv_ref, alloc_ref, alloc16_ref, et_ref):
    x = x_ref[...]
    row = lax.broadcasted_iota(jnp.int32, (16, 1), 0)
    rvalid = row < K
    ones_col = jnp.ones((N_NODES, 1), jnp.float32)
    xa = jnp.concatenate([x, ones_col], axis=1)

    def assign(c):
        cn = jnp.sum(c * c, axis=1, keepdims=True)
        b = jnp.concatenate([c, -0.5 * cn], axis=1)
        dots = lax.dot_general(b, xa, (((1,), (1,)), ((), ())),
                               preferred_element_type=jnp.float32)
        score = jnp.where(rvalid, -2.0 * dots, 1e30)
        m = jnp.min(score, axis=0, keepdims=True)
        return jnp.min(jnp.where(score == m, row, 16), axis=0,
                       keepdims=True)

    def update(cl):
        onehot = (cl == row).astype(jnp.float32)
        sa = lax.dot_general(onehot, xa, (((1,), (0,)), ((), ())),
                             preferred_element_type=jnp.float32)
        return jnp.where(rvalid, sa[:, :F] / sa[:, F:F + 1], 0.0)

    c9 = lax.fori_loop(0, NITER - 1, lambda i, c: update(assign(c)),
                       c0_ref[...])
    cl = assign(c9)
    c10 = update(cl)

    q = jnp.dot(c10, wq_ref[...], preferred_element_type=jnp.float32)
    v = jnp.dot(c10, wv_ref[...], preferred_element_type=jnp.float32)
    prods = lax.dot_general(q, v, (((1,), (1,)), ((), ())),
                            preferred_element_type=jnp.float32)
    prods = prods * (1.0 / math.sqrt(F))
    cvalid = lax.broadcasted_iota(jnp.int32, (1, 16), 1) < K
    p = jnp.where(cvalid, prods, -1e30)
    m = jnp.max(p, axis=1, keepdims=True)
    e = jnp.exp(p - m)
    tab = e / jnp.sum(e, axis=1, keepdims=True)

    alloc_ref[...] = cl
    alloc16_ref[...] = cl * 16
    et_ref[...] = jnp.exp(tab)


_tc_call = pl.pallas_call(
    _tc_body,
    out_shape=[
        jax.ShapeDtypeStruct((1, N_NODES), jnp.int32),
        jax.ShapeDtypeStruct((1, N_NODES), jnp.int32),
        jax.ShapeDtypeStruct((16, 16), jnp.float32),
    ],
)


def _sc_body(src_hbm, dst_hbm, alloc_hbm, alloc16_hbm, et_hbm, out_hbm,
             src_v, dst_v, w_v, alloc_v, alloc16_v, et_v, s_v, z_v, s_sh,
             dsem, lsem):
    sid = lax.axis_index("s")
    row0 = pl.multiple_of(sid * R_BIG, 8)
    is_last = sid == NTILES - 1
    nrows = jnp.where(is_last, R_LAST, R_BIG)

    cps = [
        pltpu.async_copy(alloc_hbm, alloc_v, lsem),
        pltpu.async_copy(alloc16_hbm, alloc16_v, lsem),
        pltpu.async_copy(et_hbm, et_v, lsem),
    ]

    @pl.when(jnp.logical_not(is_last))
    def _():
        c1 = pltpu.async_copy(src_hbm.at[pl.ds(row0, R_BIG)],
                              src_v.at[pl.ds(0, R_BIG)], lsem)
        c2 = pltpu.async_copy(dst_hbm.at[pl.ds(row0, R_BIG)],
                              dst_v.at[pl.ds(0, R_BIG)], lsem)
        c1.wait()
        c2.wait()

    @pl.when(is_last)
    def _():
        c1 = pltpu.async_copy(src_hbm.at[pl.ds(15 * R_BIG, R_LAST)],
                              src_v.at[pl.ds(0, R_LAST)], lsem)
        c2 = pltpu.async_copy(dst_hbm.at[pl.ds(15 * R_BIG, R_LAST)],
                              dst_v.at[pl.ds(0, R_LAST)], lsem)
        c1.wait()
        c2.wait()

    for t in range(ZCHUNK // 16):
        z_v[pl.ds(t * 16, 16)] = jnp.zeros((16,), jnp.float32)
    for c in cps:
        c.wait()
    pltpu.sync_copy(z_v, s_sh.at[pl.ds(sid * ZCHUNK, ZCHUNK)])
    plsc.subcore_barrier()

    def compute_row(j, carry):
        @pl.when(j >= NBUF)
        def _():
            jj = j - NBUF
            pltpu.make_async_copy(w_v.at[jj], s_sh.at[src_v.at[jj]],
                                  dsem).wait()
        for k in range(8):
            off = k * 16
            s16 = src_v[j, pl.ds(off, 16)]
            d16 = dst_v[j, pl.ds(off, 16)]
            cs16 = plsc.load_gather(alloc16_v, [s16])
            cd = plsc.load_gather(alloc_v, [d16])
            w_v[j, pl.ds(off, 16)] = plsc.load_gather(et_v, [cs16 + cd])
        pltpu.async_copy(w_v.at[j], s_sh.at[src_v.at[j]], dsem, add=True)
        return carry

    lax.fori_loop(0, nrows, compute_row, 0)

    def drain_row(j, carry):
        pltpu.make_async_copy(w_v.at[j], s_sh.at[src_v.at[j]], dsem).wait()
        return carry

    lax.fori_loop(nrows - NBUF, nrows, drain_row, 0)
    plsc.subcore_barrier()

    pltpu.sync_copy(s_sh.at[pl.ds(sid * ZCHUNK, ZCHUNK)], z_v)
    for t in range(ZCHUNK // 16):
        z_v[pl.ds(t * 16, 16)] = 1.0 / z_v[pl.ds(t * 16, 16)]
    pltpu.sync_copy(z_v, s_sh.at[pl.ds(sid * ZCHUNK, ZCHUNK)])
    plsc.subcore_barrier()
    pltpu.sync_copy(s_sh, s_v)

    def div_row(j, carry):
        for k in range(8):
            off = k * 16
            s16 = src_v[j, pl.ds(off, 16)]
            inv = plsc.load_gather(s_v, [s16])
            w_v[j, pl.ds(off, 16)] = w_v[j, pl.ds(off, 16)] * inv
        return carry

    lax.fori_loop(0, nrows, div_row, 0)

    @pl.when(jnp.logical_not(is_last))
    def _():
        pltpu.sync_copy(w_v.at[pl.ds(0, R_BIG)],
                        out_hbm.at[pl.ds(row0, R_BIG)])

    @pl.when(is_last)
    def _():
        pltpu.sync_copy(w_v.at[pl.ds(0, R_LAST)],
                        out_hbm.at[pl.ds(15 * R_BIG, R_LAST)])


_sc_call = pl.kernel(
    _sc_body,
    mesh=plsc.VectorSubcoreMesh(core_axis_name="c", subcore_axis_name="s",
                                num_cores=1),
    compiler_params=pltpu.CompilerParams(needs_layout_passes=False),
    out_type=jax.ShapeDtypeStruct((NROWS, 128), jnp.float32),
    scratch_types=[
        pltpu.VMEM((R_BIG, 128), jnp.int32),
        pltpu.VMEM((R_BIG, 128), jnp.int32),
        pltpu.VMEM((R_BIG, 128), jnp.float32),
        pltpu.VMEM((N_NODES,), jnp.int32),
        pltpu.VMEM((N_NODES,), jnp.int32),
        pltpu.VMEM((256,), jnp.float32),
        pltpu.VMEM((NPAD,), jnp.float32),
        pltpu.VMEM((ZCHUNK,), jnp.float32),
        pltpu.VMEM_SHARED((NPAD,), jnp.float32),
        pltpu.SemaphoreType.DMA,
        pltpu.SemaphoreType.DMA,
    ],
)


def kernel(x, edge, WQ, WV, WK, a):
    c0 = jnp.concatenate(
        [x[:K], jnp.zeros((16 - K, F), jnp.float32)], axis=0)
    alloc2, alloc16_2, et16 = _tc_call(x, c0, WQ, WV)
    outp = _sc_call(edge[0].reshape(NROWS, 128), edge[1].reshape(NROWS, 128),
                    alloc2.reshape(N_NODES), alloc16_2.reshape(N_NODES),
                    et16.reshape(256))
    return outp.reshape(N_EDGES)


# speedup vs baseline: 125.0101x; 1.1130x over previous
import functools
import math

import jax
import jax.numpy as jnp
from jax import lax
from jax.experimental import pallas as pl
from jax.experimental.pallas import tpu as pltpu
from jax.experimental.pallas import tpu_sc as plsc

N_NODES = 10000
N_EDGES = 320000
F = 128
K = 10
NITER = 10
KPAD = 128

NTILES = 16
NROWS = N_EDGES // 128
R_BIG = 160
R_LAST = NROWS - 15 * R_BIG
NPAD = 10240
ZCHUNK = NPAD // NTILES
NBUF = 16


def _tc_body(x_ref, c0_ref, wq_ref, wv_ref, alloc_ref, alloc16_ref, et_ref):
    x = x_ref[...]
    row = lax.broadcasted_iota(jnp.int32, (16, 1), 0)
    rvalid = row < K
    ones_col = jnp.ones((N_NODES, 1), jnp.float32)
    xa = jnp.concatenate([x, ones_col], axis=1)

    def assign(c):
        cn = jnp.sum(c * c, axis=1, keepdims=True)
        b = jnp.concatenate([c, -0.5 * cn], axis=1)
        dots = lax.dot_general(b, xa, (((1,), (1,)), ((), ())),
                               preferred_element_type=jnp.float32)
        score = jnp.where(rvalid, -2.0 * dots, 1e30)
        m = jnp.min(score, axis=0, keepdims=True)
        return jnp.min(jnp.where(score == m, row, 16), axis=0,
                       keepdims=True)

    def update(cl):
        onehot = (cl == row).astype(jnp.float32)
        sa = lax.dot_general(onehot, xa, (((1,), (0,)), ((), ())),
                             preferred_element_type=jnp.float32)
        return jnp.where(rvalid, sa[:, :F] / sa[:, F:F + 1], 0.0)

    c9 = lax.fori_loop(0, NITER - 1, lambda i, c: update(assign(c)),
                       c0_ref[...])
    cl = assign(c9)
    c10 = update(cl)

    q = jnp.dot(c10, wq_ref[...], preferred_element_type=jnp.float32)
    v = jnp.dot(c10, wv_ref[...], preferred_element_type=jnp.float32)
    prods = lax.dot_general(q, v, (((1,), (1,)), ((), ())),
                            preferred_element_type=jnp.float32)
    prods = prods * (1.0 / math.sqrt(F))
    cvalid = lax.broadcasted_iota(jnp.int32, (1, 16), 1) < K
    p = jnp.where(cvalid, prods, -1e30)
    m = jnp.max(p, axis=1, keepdims=True)
    e = jnp.exp(p - m)
    tab = e / jnp.sum(e, axis=1, keepdims=True)

    alloc_ref[...] = cl
    alloc16_ref[...] = cl * 16
    et_ref[...] = jnp.exp(tab)


_tc_call = pl.pallas_call(
    _tc_body,
    out_shape=[
        jax.ShapeDtypeStruct((1, N_NODES), jnp.int32),
        jax.ShapeDtypeStruct((1, N_NODES), jnp.int32),
        jax.ShapeDtypeStruct((16, 16), jnp.float32),
    ],
)


def _sc_body(src_hbm, dst_hbm, alloc_hbm, alloc16_hbm, et_hbm, out_hbm,
             src_v, dst_v, w_v, alloc_v, alloc16_v, et_v, s_v, z_v, s_sh,
             dsem, lsem):
    sid = lax.axis_index("s")
    row0 = pl.multiple_of(sid * R_BIG, 8)
    is_last = sid == NTILES - 1
    nrows = jnp.where(is_last, R_LAST, R_BIG)

    cps = [
        pltpu.async_copy(alloc_hbm, alloc_v, lsem),
        pltpu.async_copy(alloc16_hbm, alloc16_v, lsem),
        pltpu.async_copy(et_hbm, et_v, lsem),
    ]

    @pl.when(jnp.logical_not(is_last))
    def _():
        c1 = pltpu.async_copy(src_hbm.at[pl.ds(row0, R_BIG)],
                              src_v.at[pl.ds(0, R_BIG)], lsem)
        c2 = pltpu.async_copy(dst_hbm.at[pl.ds(row0, R_BIG)],
                              dst_v.at[pl.ds(0, R_BIG)], lsem)
        c1.wait()
        c2.wait()

    @pl.when(is_last)
    def _():
        c1 = pltpu.async_copy(src_hbm.at[pl.ds(15 * R_BIG, R_LAST)],
                              src_v.at[pl.ds(0, R_LAST)], lsem)
        c2 = pltpu.async_copy(dst_hbm.at[pl.ds(15 * R_BIG, R_LAST)],
                              dst_v.at[pl.ds(0, R_LAST)], lsem)
        c1.wait()
        c2.wait()

    for t in range(ZCHUNK // 16):
        z_v[pl.ds(t * 16, 16)] = jnp.zeros((16,), jnp.float32)
    for c in cps:
        c.wait()
    pltpu.sync_copy(z_v, s_sh.at[pl.ds(sid * ZCHUNK, ZCHUNK)])
    plsc.subcore_barrier()

    def fill_row(j):
        for k in range(8):
            off = k * 16
            s16 = src_v[j, pl.ds(off, 16)]
            d16 = dst_v[j, pl.ds(off, 16)]
            cs16 = plsc.load_gather(alloc16_v, [s16])
            cd = plsc.load_gather(alloc_v, [d16])
            w_v[j, pl.ds(off, 16)] = plsc.load_gather(et_v, [cs16 + cd])

    def fire(j):
        pltpu.async_copy(w_v.at[j], s_sh.at[src_v.at[j]], dsem, add=True)

    def wait(j):
        pltpu.make_async_copy(w_v.at[j], s_sh.at[src_v.at[j]], dsem).wait()

    def prologue_row(j, carry):
        fill_row(j)
        fire(j)
        return carry

    def steady_row(j, carry):
        wait(j - NBUF)
        fill_row(j)
        fire(j)
        return carry

    def drain_row(j, carry):
        wait(j)
        return carry

    lax.fori_loop(0, NBUF, prologue_row, 0)
    lax.fori_loop(NBUF, nrows, steady_row, 0)
    lax.fori_loop(nrows - NBUF, nrows, drain_row, 0)
    plsc.subcore_barrier()

    pltpu.sync_copy(s_sh.at[pl.ds(sid * ZCHUNK, ZCHUNK)], z_v)
    for t in range(ZCHUNK // 16):
        z_v[pl.ds(t * 16, 16)] = 1.0 / z_v[pl.ds(t * 16, 16)]
    pltpu.sync_copy(z_v, s_sh.at[pl.ds(sid * ZCHUNK, ZCHUNK)])
    plsc.subcore_barrier()
    pltpu.sync_copy(s_sh, s_v)

    @plsc.parallel_loop(0, nrows, 1, unroll=2)
    def _(j):
        for k in range(8):
            off = k * 16
            s16 = src_v[j, pl.ds(off, 16)]
            inv = plsc.load_gather(s_v, [s16])
            w_v[j, pl.ds(off, 16)] = w_v[j, pl.ds(off, 16)] * inv

    @pl.when(jnp.logical_not(is_last))
    def _():
        pltpu.sync_copy(w_v.at[pl.ds(0, R_BIG)],
                        out_hbm.at[pl.ds(row0, R_BIG)])

    @pl.when(is_last)
    def _():
        pltpu.sync_copy(w_v.at[pl.ds(0, R_LAST)],
                        out_hbm.at[pl.ds(15 * R_BIG, R_LAST)])


_sc_call = pl.kernel(
    _sc_body,
    mesh=plsc.VectorSubcoreMesh(core_axis_name="c", subcore_axis_name="s",
                                num_cores=1),
    compiler_params=pltpu.CompilerParams(needs_layout_passes=False),
    out_type=jax.ShapeDtypeStruct((NROWS, 128), jnp.float32),
    scratch_types=[
        pltpu.VMEM((R_BIG, 128), jnp.int32),
        pltpu.VMEM((R_BIG, 128), jnp.int32),
        pltpu.VMEM((R_BIG, 128), jnp.float32),
        pltpu.VMEM((N_NODES,), jnp.int32),
        pltpu.VMEM((N_NODES,), jnp.int32),
        pltpu.VMEM((256,), jnp.float32),
        pltpu.VMEM((NPAD,), jnp.float32),
        pltpu.VMEM((ZCHUNK,), jnp.float32),
        pltpu.VMEM_SHARED((NPAD,), jnp.float32),
        pltpu.SemaphoreType.DMA,
        pltpu.SemaphoreType.DMA,
    ],
)


def kernel(x, edge, WQ, WV, WK, a):
    c0 = jnp.concatenate(
        [x[:K], jnp.zeros((16 - K, F), jnp.float32)], axis=0)
    alloc2, alloc16_2, et16 = _tc_call(x, c0, WQ, WV)
    outp = _sc_call(edge[0].reshape(NROWS, 128), edge[1].reshape(NROWS, 128),
                    alloc2.reshape(N_NODES), alloc16_2.reshape(N_NODES),
                    et16.reshape(256))
    return outp.reshape(N_EDGES)
